# Initial kernel scaffold; baseline (speedup 1.0000x reference)
import jax
import jax.numpy as jnp
from jax.experimental import pallas as pl


def kernel(x, table):
    raise NotImplementedError("write your pallas kernel here")


# speedup vs baseline: 1.3053x; 1.3053x over previous
import functools

import jax
import jax.numpy as jnp
from jax import lax
from jax.experimental import pallas as pl
from jax.experimental.pallas import tpu as pltpu
from jax.experimental.pallas import tpu_sc as plsc

VOCAB = 1000000
DIM = 32
BATCH = 4096
SEQ = 200

CHUNK = 128
TOTAL = BATCH * SEQ
NUM_CHUNKS = TOTAL // CHUNK


@functools.partial(jax.jit, static_argnums=())
def _embed(table, idx2d):
    info = plsc.get_sparse_core_info()
    nw = info.num_cores * info.num_subcores
    chunks_per_w = NUM_CHUNKS // nw

    mesh = plsc.VectorSubcoreMesh(core_axis_name="c", subcore_axis_name="s")

    @functools.partial(
        pl.kernel,
        mesh=mesh,
        out_type=jax.ShapeDtypeStruct((TOTAL, DIM), jnp.float32),
        scratch_types=[
            pltpu.VMEM((chunks_per_w, CHUNK), jnp.int32),
            pltpu.VMEM((CHUNK, DIM), jnp.float32),
            pltpu.SemaphoreType.DMA,
        ],
        compiler_params=pltpu.CompilerParams(use_tc_tiling_on_sc=False),
    )
    def k(idx_hbm, table_hbm, out_hbm, idx_v, rows_v, sem):
        wid = lax.axis_index("s") * info.num_cores + lax.axis_index("c")
        cbase = wid * chunks_per_w
        pltpu.sync_copy(idx_hbm.at[pl.ds(cbase, chunks_per_w)], idx_v)

        def chunk_body(j, _):
            pltpu.async_copy(table_hbm.at[idx_v.at[j]], rows_v, sem).wait()
            pltpu.sync_copy(
                rows_v, out_hbm.at[pl.ds((cbase + j) * CHUNK, CHUNK)])
            return 0

        lax.fori_loop(0, chunks_per_w, chunk_body, 0)

    return k(idx2d, table)


def kernel(x, table):
    idx2d = x.reshape(NUM_CHUNKS, CHUNK)
    out = _embed(table, idx2d)
    return out.reshape(BATCH, SEQ, DIM)


# speedup vs baseline: 1.5000x; 1.1492x over previous
import functools

import jax
import jax.numpy as jnp
from jax import lax
from jax.experimental import pallas as pl
from jax.experimental.pallas import tpu as pltpu
from jax.experimental.pallas import tpu_sc as plsc

VOCAB = 1000000
DIM = 32
BATCH = 4096
SEQ = 200

CHUNK = 128
TOTAL = BATCH * SEQ
NUM_CHUNKS = TOTAL // CHUNK

K = 5
NBUF = 4
GROW = K * CHUNK


@jax.jit
def _embed(table, idx2d):
    info = plsc.get_sparse_core_info()
    nw = info.num_cores * info.num_subcores
    chunks_per_w = NUM_CHUNKS // nw
    groups = chunks_per_w // K

    mesh = plsc.VectorSubcoreMesh(core_axis_name="c", subcore_axis_name="s")

    @functools.partial(
        pl.kernel,
        mesh=mesh,
        out_type=jax.ShapeDtypeStruct((TOTAL, DIM), jnp.float32),
        scratch_types=[
            pltpu.VMEM((chunks_per_w, CHUNK), jnp.int32),
            pltpu.VMEM((NBUF, GROW, DIM), jnp.float32),
            pltpu.SemaphoreType.DMA((NBUF,)),
            pltpu.SemaphoreType.DMA((NBUF,)),
        ],
        compiler_params=pltpu.CompilerParams(use_tc_tiling_on_sc=False),
    )
    def k(idx_hbm, table_hbm, out_hbm, idx_v, rows_v, gsem, wsem):
        wid = lax.axis_index("s") * info.num_cores + lax.axis_index("c")
        cbase = wid * chunks_per_w
        rbase = cbase * CHUNK
        pltpu.sync_copy(idx_hbm.at[pl.ds(cbase, chunks_per_w)], idx_v)

        def fire(g, b):
            for j in range(K):
                pltpu.async_copy(
                    table_hbm.at[idx_v.at[g * K + j]],
                    rows_v.at[b, pl.ds(j * CHUNK, CHUNK)],
                    gsem.at[b])

        def drain_gather(b):
            pltpu.make_async_copy(
                out_hbm.at[pl.ds(0, GROW)], rows_v.at[b], gsem.at[b]).wait()

        def write(g, b):
            pltpu.async_copy(
                rows_v.at[b], out_hbm.at[pl.ds(rbase + g * GROW, GROW)],
                wsem.at[b])

        def wait_write(b):
            pltpu.make_async_copy(
                rows_v.at[b], out_hbm.at[pl.ds(0, GROW)], wsem.at[b]).wait()

        fire(0, 0)
        fire(1, 1)
        fire(2, 2)
        fire(3, 3)
        drain_gather(0)
        write(0, 0)
        drain_gather(1)
        write(1, 1)

        def body(i, _):
            for b in range(NBUF):
                g = 2 + i * NBUF + b
                bu = (2 + b) % NBUF
                wait_write(b)
                fire(g + 2, b)
                drain_gather(bu)
                write(g, bu)
            return 0

        lax.fori_loop(0, (groups - 4) // NBUF, body, 0)

        drain_gather((groups - 2) % NBUF)
        write(groups - 2, (groups - 2) % NBUF)
        drain_gather((groups - 1) % NBUF)
        write(groups - 1, (groups - 1) % NBUF)
        for b in range(NBUF):
            wait_write(b)

    return k(idx2d, table)


def kernel(x, table):
    idx2d = x.reshape(NUM_CHUNKS, CHUNK)
    out = _embed(table, idx2d)
    return out.reshape(BATCH, SEQ, DIM)
